# baseline (speedup 1.0000x reference)
import functools

import jax
import jax.numpy as jnp
from jax import lax
from jax.experimental import pallas as pl
from jax.experimental.pallas import tpu as pltpu
from jax.experimental.pallas import tpu_sc as plsc

D = 128
BT = 4096
T = 50
TP = 56
NC, NS = 2, 16
NW = NC * NS
NBW = BT // NW
NBUF = 8

_mesh = plsc.VectorSubcoreMesh(core_axis_name="c", subcore_axis_name="s")


@functools.partial(
    pl.kernel,
    out_type=jax.ShapeDtypeStruct((BT, TP, D), jnp.float32),
    mesh=_mesh,
    scratch_types=[
        pltpu.VMEM((NBW * TP,), jnp.int32),
        pltpu.VMEM((NBUF, TP, D), jnp.float32),
        pltpu.SemaphoreType.DMA,
        pltpu.SemaphoreType.DMA,
        pltpu.SemaphoreType.DMA,
        pltpu.SemaphoreType.DMA,
        pltpu.SemaphoreType.DMA,
        pltpu.SemaphoreType.DMA,
        pltpu.SemaphoreType.DMA,
        pltpu.SemaphoreType.DMA,
        pltpu.SemaphoreType.DMA,
        pltpu.SemaphoreType.DMA,
        pltpu.SemaphoreType.DMA,
        pltpu.SemaphoreType.DMA,
        pltpu.SemaphoreType.DMA,
        pltpu.SemaphoreType.DMA,
        pltpu.SemaphoreType.DMA,
        pltpu.SemaphoreType.DMA,
    ],
)
def _gather_kernel(idx_hbm, table_hbm, out_hbm, idx_v, bufs, *sems):
    sg = sems[:NBUF]
    so = sems[NBUF:]
    wid = lax.axis_index("s") * NC + lax.axis_index("c")
    wb = wid * NBW
    pltpu.sync_copy(idx_hbm.at[wid], idx_v)

    def gather_start(b, i):
        pltpu.make_async_copy(
            table_hbm.at[idx_v.at[pl.ds(b * TP, TP)]],
            bufs.at[i], sg[i]).start()

    def gather_wait(i):
        pltpu.make_async_copy(
            table_hbm.at[idx_v.at[pl.ds(0, TP)]],
            bufs.at[i], sg[i]).wait()

    def store_start(b, i):
        pltpu.make_async_copy(bufs.at[i], out_hbm.at[wb + b], so[i]).start()

    def store_wait(b, i):
        pltpu.make_async_copy(bufs.at[i], out_hbm.at[wb + b], so[i]).wait()

    H = NBUF // 2

    def step(b, s, prefetch=True, first=False):
        gather_wait(s)
        store_start(b, s)
        if prefetch:
            ps = (s + H) % NBUF
            if not first:
                store_wait(b + H - NBUF, ps)
            gather_start(b + H, ps)

    for i in range(H):
        gather_start(i, i)
    for b in range(NBUF - H):
        step(b, b % NBUF, first=True)

    def body(q, carry):
        for i in range(NBUF):
            step(NBUF - H + q * NBUF + i, (NBUF - H + i) % NBUF)
        return carry

    lax.fori_loop(0, (NBW - NBUF) // NBUF, body, 0)

    for b in range(NBW - H, NBW):
        step(b, b % NBUF, prefetch=False)
    for b in range(NBW - NBUF, NBW):
        store_wait(b, b % NBUF)


def kernel(token_ids, weight):
    idx = jnp.pad(token_ids.astype(jnp.int32), ((0, 0), (0, TP - T)))
    return _gather_kernel(idx.reshape(NW, NBW * TP), weight)[:, :T, :]


# speedup vs baseline: 4.4326x; 4.4326x over previous
import functools

import jax
import jax.numpy as jnp
from jax import lax
from jax.experimental import pallas as pl
from jax.experimental.pallas import tpu as pltpu
from jax.experimental.pallas import tpu_sc as plsc

D = 128
B = 4096 * 50
NC, NS = 2, 16
NW = NC * NS
BPW = B // NW
C = 128
J = BPW // C
K = 3
NFULL = J // K
RK = J - NFULL * K

_mesh = plsc.VectorSubcoreMesh(core_axis_name="c", subcore_axis_name="s")


@functools.partial(
    pl.kernel,
    out_type=jax.ShapeDtypeStruct((B, D), jnp.float32),
    mesh=_mesh,
    scratch_types=[
        pltpu.VMEM((J, C), jnp.int32),
        pltpu.VMEM((K * C, D), jnp.float32),
        pltpu.VMEM((K * C, D), jnp.float32),
        pltpu.SemaphoreType.DMA,
        pltpu.SemaphoreType.DMA,
        pltpu.SemaphoreType.DMA,
        pltpu.SemaphoreType.DMA,
    ],
)
def _gather_kernel(idx_hbm, table_hbm, out_hbm,
                   idx_v, r0, r1, g0, g1, o0, o1):
    rows = (r0, r1)
    sg = (g0, g1)
    so = (o0, o1)
    wid = lax.axis_index("s") * NC + lax.axis_index("c")
    base = wid * BPW
    pltpu.sync_copy(idx_hbm.at[wid], idx_v)

    def gather_start(r, b, k=K):
        for i in range(k):
            pltpu.make_async_copy(table_hbm.at[idx_v.at[r * K + i]],
                                  rows[b].at[pl.ds(i * C, C)], sg[b]).start()

    def gather_wait(b, k=K):
        pltpu.make_async_copy(table_hbm.at[idx_v.at[0]],
                              rows[b].at[pl.ds(0, k * C)], sg[b]).wait()

    def out_start(r, b, k=K):
        pltpu.make_async_copy(rows[b].at[pl.ds(0, k * C)],
                              out_hbm.at[pl.ds(base + r * K * C, k * C)],
                              so[b]).start()

    def out_wait(r, b, k=K):
        pltpu.make_async_copy(rows[b].at[pl.ds(0, k * C)],
                              out_hbm.at[pl.ds(base + r * K * C, k * C)],
                              so[b]).wait()

    gather_start(0, 0)
    gather_start(1, 1)
    gather_wait(0)
    out_start(0, 0)

    def body(g, carry):
        for b in range(2):
            r = 1 + g * 2 + b
            out_wait(r - 1, b)
            gather_start(r + 1, b)
            gather_wait(1 - b)
            out_start(r, 1 - b)
        return carry

    lax.fori_loop(0, (NFULL - 2) // 2, body, 0)

    out_wait(NFULL - 2, 0)
    gather_start(NFULL, 0, k=RK)
    gather_wait(1)
    out_start(NFULL - 1, 1)
    out_wait(NFULL - 1, 1)
    gather_wait(0, k=RK)
    out_start(NFULL, 0, k=RK)
    out_wait(NFULL, 0, k=RK)


def kernel(token_ids, weight):
    idx = token_ids.reshape(NW, J, C).astype(jnp.int32)
    out = _gather_kernel(idx, weight)
    return out.reshape(4096, 50, D)
